# baseline (speedup 1.0000x reference)
import functools
import math

import jax
import jax.numpy as jnp
from jax import lax
from jax.experimental import pallas as pl
from jax.experimental.pallas import tpu as pltpu
from jax.experimental.pallas import tpu_sc as plsc

B = 16384
J = 50
V = 1000000
D = 32
SCALE = math.sqrt(float(D))
NC = 2
NS = 16
NW = NC * NS
CHUNK = B // NW
HALF = CHUNK // 2
NSTEP = 2 * J
LANES = 16

_mesh = plsc.VectorSubcoreMesh(core_axis_name="c", subcore_axis_name="s")


@functools.partial(
    pl.kernel,
    mesh=_mesh,
    out_type=jax.ShapeDtypeStruct((J, D // 8, B // 128, 8, 128), jnp.float32),
    compiler_params=pltpu.CompilerParams(
        use_tc_tiling_on_sc=False, needs_layout_passes=False
    ),
    scratch_types=[
        pltpu.VMEM((J, CHUNK), jnp.int32),
        pltpu.VMEM((HALF,), jnp.int32),
        pltpu.VMEM((HALF,), jnp.int32),
        pltpu.VMEM((HALF, 4 * D), jnp.float32),
        pltpu.VMEM((HALF, 4 * D), jnp.float32),
        pltpu.VMEM((D // 8, CHUNK // 128, 8, 130), jnp.float32),
        pltpu.SemaphoreType.DMA,
        pltpu.SemaphoreType.DMA,
    ],
)
def _emb_lookup(
    tok_hbm, tab4_hbm, out_hbm, idx_v, q0, q1, rows0, rows1, tile_v, sem0, sem1
):
    wid = lax.axis_index("s") * NC + lax.axis_index("c")
    b0 = wid * CHUNK
    pltpu.sync_copy(tok_hbm.at[:, pl.ds(b0, CHUNK)], idx_v)

    bufs = (rows0, rows1)
    qrefs = (q0, q1)
    sems = (sem0, sem1)
    lv = lax.broadcasted_iota(jnp.int32, (LANES,), 0)
    r_lo = lv >> 3
    s_all = lv & 7

    def stage_q(s, b):
        j = s >> 1
        off = (s & 1) * HALF

        def q_body(g, carry):
            qrefs[b][pl.ds(g * LANES, LANES)] = (
                idx_v[j, pl.ds(off + g * LANES, LANES)] >> 2
            )
            return carry

        lax.fori_loop(0, HALF // LANES, q_body, 0)

    def gather(b):
        return pltpu.async_copy(tab4_hbm.at[qrefs[b]], bufs[b], sems[b])

    def transform(buf, s):
        j = s >> 1
        off = (s & 1) * HALF

        def t_body(g, carry):
            mv = (idx_v[j, pl.ds(off + g * LANES, LANES)] & 3) * D
            for u in range(LANES):
                i = g * LANES + u
                ig = off + i
                m = mv[u]
                cs = jnp.full((LANES,), ig >> 7, jnp.int32)
                rs = jnp.full((LANES,), ig & 127, jnp.int32)
                for h in (0, 16):
                    col = m + h + lv
                    vec = plsc.load_gather(buf, [jnp.full((LANES,), i, jnp.int32), col])
                    plsc.store_scatter(
                        tile_v, [r_lo + (h // 8), cs, s_all, rs], vec * SCALE
                    )
            return carry

        lax.fori_loop(0, HALF // LANES, t_body, 0)

    stage_q(0, 0)
    gather(0)

    def pair_body(p, carry):
        for b in (0, 1):
            s = 2 * p + b

            @pl.when(s + 1 < NSTEP)
            def _():
                stage_q(s + 1, 1 - b)
                gather(1 - b)

            pltpu.make_async_copy(tab4_hbm.at[qrefs[b]], bufs[b], sems[b]).wait()
            transform(bufs[b], s)

            @pl.when((s & 1) == 1)
            def _():
                pltpu.sync_copy(
                    tile_v.at[:, :, :, pl.ds(0, 128)],
                    out_hbm.at[s >> 1, :, pl.ds(wid * (CHUNK // 128), CHUNK // 128)],
                )
        return carry

    lax.fori_loop(0, NSTEP // 2, pair_body, 0)


def kernel(tokens, table):
    tok_t = tokens.T.astype(jnp.int32)
    rows4 = table.reshape(V // 4, 4 * D)
    out5 = _emb_lookup(tok_t, rows4)
    return out5.transpose(2, 4, 0, 1, 3).reshape(B, J, D)


# speedup vs baseline: 1.0697x; 1.0697x over previous
import functools
import math

import jax
import jax.numpy as jnp
from jax import lax
from jax.experimental import pallas as pl
from jax.experimental.pallas import tpu as pltpu
from jax.experimental.pallas import tpu_sc as plsc

B = 16384
J = 50
V = 1000000
D = 32
SCALE = math.sqrt(float(D))
NC = 2
NS = 16
NW = NC * NS
CHUNK = B // NW
LANES = 16

_mesh = plsc.VectorSubcoreMesh(core_axis_name="c", subcore_axis_name="s")


@functools.partial(
    pl.kernel,
    mesh=_mesh,
    out_type=jax.ShapeDtypeStruct((J, D // 8, B // 128, 8, 128), jnp.float32),
    compiler_params=pltpu.CompilerParams(
        use_tc_tiling_on_sc=False, needs_layout_passes=False
    ),
    scratch_types=[
        pltpu.VMEM((J, CHUNK), jnp.int32),
        pltpu.VMEM((CHUNK, D), jnp.float32),
        pltpu.VMEM((CHUNK, D), jnp.float32),
        pltpu.VMEM((D // 8, CHUNK // 128, 8, 130), jnp.float32),
        pltpu.SemaphoreType.DMA,
        pltpu.SemaphoreType.DMA,
    ],
)
def _emb_lookup(tok_hbm, table_hbm, out_hbm, idx_v, rows0, rows1, tile_v, sem0, sem1):
    wid = lax.axis_index("s") * NC + lax.axis_index("c")
    b0 = wid * CHUNK
    pltpu.sync_copy(tok_hbm.at[:, pl.ds(b0, CHUNK)], idx_v)

    bufs = (rows0, rows1)
    sems = (sem0, sem1)

    def gather(j, b):
        return pltpu.async_copy(table_hbm.at[idx_v.at[j]], bufs[b], sems[b])

    lv = lax.broadcasted_iota(jnp.int32, (LANES,), 0)
    r_lo = lv >> 3
    s_all = lv & 7

    def transform(buf):
        def t_body(i4, carry):
            for u in range(4):
                i = i4 * 4 + u
                cs = jnp.full((LANES,), i >> 7, jnp.int32)
                rs = jnp.full((LANES,), i & 127, jnp.int32)
                for h in (0, 16):
                    vec = buf[i, pl.ds(h, LANES)]
                    plsc.store_scatter(
                        tile_v, [r_lo + (h // 8), cs, s_all, rs], vec * SCALE
                    )
            return carry

        lax.fori_loop(0, CHUNK // 4, t_body, 0)

    gather(0, 0)

    def pair_body(p, carry):
        for b in (0, 1):
            j = 2 * p + b

            @pl.when(j + 1 < J)
            def _():
                gather(j + 1, 1 - b)

            pltpu.make_async_copy(
                table_hbm.at[idx_v.at[j]], bufs[b], sems[b]
            ).wait()
            transform(bufs[b])
            pltpu.sync_copy(
                tile_v.at[:, :, :, pl.ds(0, 128)],
                out_hbm.at[j, :, pl.ds(wid * (CHUNK // 128), CHUNK // 128)],
            )
        return carry

    lax.fori_loop(0, J // 2, pair_body, 0)


def kernel(tokens, table):
    tok_t = tokens.T.astype(jnp.int32)
    out5 = _emb_lookup(tok_t, table)
    return out5.transpose(2, 4, 0, 1, 3).reshape(B, J, D)
